# Initial kernel scaffold; baseline (speedup 1.0000x reference)
import jax
import jax.numpy as jnp
from jax.experimental import pallas as pl


def kernel(X, conv_weight):
    raise NotImplementedError("write your pallas kernel here")


# speedup vs baseline: 3.5605x; 3.5605x over previous
import functools

import jax
import jax.numpy as jnp
from jax.experimental import pallas as pl
from jax.experimental.pallas import tpu as pltpu


def _build_patches(x_ref, p_ref, *, cin, kh, kw, w_img, n_lanes, hw):
    for ikh in range(kh):
        for ikw in range(kw):
            t = ikh * kw + ikw
            off = ikh * w_img + ikw
            n = min(n_lanes, hw - off)
            p_ref[t * cin:(t + 1) * cin, :n] = (
                x_ref[0, :, off:off + n].astype(jnp.bfloat16))


def _conv_stats_kernel(w_ref, x_ref, stats_ref, p_ref, *,
                       cin, kh, kw, w_img, wo, n_lanes, hw):
    _build_patches(x_ref, p_ref, cin=cin, kh=kh, kw=kw, w_img=w_img,
                   n_lanes=n_lanes, hw=hw)
    y = jnp.dot(w_ref[...], p_ref[...], preferred_element_type=jnp.float32)
    lane = jax.lax.broadcasted_iota(jnp.int32, (1, n_lanes), 1)
    ym = jnp.where(lane % w_img < wo, y, 0.0)
    stats_ref[0, :, 0:1] = jnp.sum(ym, axis=1, keepdims=True)
    stats_ref[0, :, 1:2] = jnp.sum(ym * ym, axis=1, keepdims=True)


def _conv_norm_kernel(w_ref, stats_ref, x_ref, o_ref, p_ref, *,
                      cin, kh, kw, w_img, wo, ho, n_lanes, hw, count, eps):
    _build_patches(x_ref, p_ref, cin=cin, kh=kh, kw=kw, w_img=w_img,
                   n_lanes=n_lanes, hw=hw)
    y = jnp.dot(w_ref[...], p_ref[...], preferred_element_type=jnp.float32)
    st = jnp.sum(stats_ref[...], axis=0)
    s = st[:, 0:1]
    ss = st[:, 1:2]
    mean = s * (1.0 / count)
    var = (ss - s * mean) * (1.0 / (count - 1.0))
    inv = 1.0 / (jnp.sqrt(var) + eps)
    o = (y - mean) * inv
    for r in range(ho):
        o_ref[0, :, r, :] = o[:, r * w_img:r * w_img + wo]


def kernel(X, conv_weight):
    n, cin, h, w_img = X.shape
    cout, _, kh, kw = conv_weight.shape
    ho = h - kh + 1
    wo = w_img - kw + 1
    hw = h * w_img
    n_lanes = ho * w_img
    k_dim = cin * kh * kw
    count = float(n * ho * wo)
    eps = 1.0

    x3 = X.reshape(n, cin, hw)
    w_mat = (conv_weight.transpose(0, 2, 3, 1)
             .reshape(cout, k_dim).astype(jnp.bfloat16))

    vmem_limit = 48 * 1024 * 1024

    stats = pl.pallas_call(
        functools.partial(_conv_stats_kernel, cin=cin, kh=kh, kw=kw,
                          w_img=w_img, wo=wo, n_lanes=n_lanes, hw=hw),
        out_shape=jax.ShapeDtypeStruct((n, cout, 2), jnp.float32),
        grid=(n,),
        in_specs=[pl.BlockSpec((cout, k_dim), lambda i: (0, 0)),
                  pl.BlockSpec((1, cin, hw), lambda i: (i, 0, 0))],
        out_specs=pl.BlockSpec((1, cout, 2), lambda i: (i, 0, 0)),
        scratch_shapes=[pltpu.VMEM((k_dim, n_lanes), jnp.bfloat16)],
        compiler_params=pltpu.CompilerParams(
            dimension_semantics=("parallel",),
            vmem_limit_bytes=vmem_limit),
    )(w_mat, x3)

    out = pl.pallas_call(
        functools.partial(_conv_norm_kernel, cin=cin, kh=kh, kw=kw,
                          w_img=w_img, wo=wo, ho=ho, n_lanes=n_lanes, hw=hw,
                          count=count, eps=eps),
        out_shape=jax.ShapeDtypeStruct((n, cout, ho, wo), jnp.float32),
        grid=(n,),
        in_specs=[pl.BlockSpec((cout, k_dim), lambda i: (0, 0)),
                  pl.BlockSpec((n, cout, 2), lambda i: (0, 0, 0)),
                  pl.BlockSpec((1, cin, hw), lambda i: (i, 0, 0))],
        out_specs=pl.BlockSpec((1, cout, ho, wo), lambda i: (i, 0, 0, 0)),
        scratch_shapes=[pltpu.VMEM((k_dim, n_lanes), jnp.bfloat16)],
        compiler_params=pltpu.CompilerParams(
            dimension_semantics=("parallel",),
            vmem_limit_bytes=vmem_limit),
    )(w_mat, stats, x3)
    return out


# speedup vs baseline: 7.3465x; 2.0634x over previous
import functools

import jax
import jax.numpy as jnp
from jax.experimental import pallas as pl
from jax.experimental.pallas import tpu as pltpu


def _build_patches(x_ref, p_ref, *, cin, kh, kw, w_img, n_lanes, hw):
    for ikh in range(kh):
        for ikw in range(kw):
            t = ikh * kw + ikw
            off = ikh * w_img + ikw
            n = min(n_lanes, hw - off)
            p_ref[t * cin:(t + 1) * cin, :n] = (
                x_ref[0, :, off:off + n].astype(jnp.bfloat16))


def _conv_stats_kernel(w_ref, x_ref, stats_ref, p_ref, *,
                       cin, kh, kw, w_img, wo, n_lanes, hw):
    _build_patches(x_ref, p_ref, cin=cin, kh=kh, kw=kw, w_img=w_img,
                   n_lanes=n_lanes, hw=hw)
    y = jnp.dot(w_ref[...], p_ref[...], preferred_element_type=jnp.float32)
    lane = jax.lax.broadcasted_iota(jnp.int32, (1, n_lanes), 1)
    ym = jnp.where(lane % w_img < wo, y, 0.0)
    stats_ref[0, :, 0:1] = jnp.sum(ym, axis=1, keepdims=True)
    stats_ref[0, :, 1:2] = jnp.sum(ym * ym, axis=1, keepdims=True)


def _conv_norm_kernel(w_ref, stats_ref, x_ref, o_ref, p_ref, *,
                      cin, kh, kw, w_img, wo, ho, n_lanes, hw, count, eps):
    _build_patches(x_ref, p_ref, cin=cin, kh=kh, kw=kw, w_img=w_img,
                   n_lanes=n_lanes, hw=hw)
    y = jnp.dot(w_ref[...], p_ref[...], preferred_element_type=jnp.float32)
    st = jnp.sum(stats_ref[...], axis=0)
    s = st[:, 0:1]
    ss = st[:, 1:2]
    mean = s * (1.0 / count)
    var = (ss - s * mean) * (1.0 / (count - 1.0))
    inv = 1.0 / (jnp.sqrt(var) + eps)
    o_ref[0] = (y - mean) * inv


def kernel(X, conv_weight):
    n, cin, h, w_img = X.shape
    cout, _, kh, kw = conv_weight.shape
    ho = h - kh + 1
    wo = w_img - kw + 1
    hw = h * w_img
    n_lanes = ho * w_img
    k_dim = cin * kh * kw
    count = float(n * ho * wo)
    eps = 1.0

    x3 = X.reshape(n, cin, hw)
    w_mat = (conv_weight.transpose(0, 2, 3, 1)
             .reshape(cout, k_dim).astype(jnp.bfloat16))

    vmem_limit = 48 * 1024 * 1024

    stats = pl.pallas_call(
        functools.partial(_conv_stats_kernel, cin=cin, kh=kh, kw=kw,
                          w_img=w_img, wo=wo, n_lanes=n_lanes, hw=hw),
        out_shape=jax.ShapeDtypeStruct((n, cout, 2), jnp.float32),
        grid=(n,),
        in_specs=[pl.BlockSpec((cout, k_dim), lambda i: (0, 0)),
                  pl.BlockSpec((1, cin, hw), lambda i: (i, 0, 0))],
        out_specs=pl.BlockSpec((1, cout, 2), lambda i: (i, 0, 0)),
        scratch_shapes=[pltpu.VMEM((k_dim, n_lanes), jnp.bfloat16)],
        compiler_params=pltpu.CompilerParams(
            dimension_semantics=("parallel",),
            vmem_limit_bytes=vmem_limit),
    )(w_mat, x3)

    out_pad = pl.pallas_call(
        functools.partial(_conv_norm_kernel, cin=cin, kh=kh, kw=kw,
                          w_img=w_img, wo=wo, ho=ho, n_lanes=n_lanes, hw=hw,
                          count=count, eps=eps),
        out_shape=jax.ShapeDtypeStruct((n, cout, n_lanes), jnp.float32),
        grid=(n,),
        in_specs=[pl.BlockSpec((cout, k_dim), lambda i: (0, 0)),
                  pl.BlockSpec((n, cout, 2), lambda i: (0, 0, 0)),
                  pl.BlockSpec((1, cin, hw), lambda i: (i, 0, 0))],
        out_specs=pl.BlockSpec((1, cout, n_lanes), lambda i: (i, 0, 0)),
        scratch_shapes=[pltpu.VMEM((k_dim, n_lanes), jnp.bfloat16)],
        compiler_params=pltpu.CompilerParams(
            dimension_semantics=("parallel",),
            vmem_limit_bytes=vmem_limit),
    )(w_mat, stats, x3)
    return out_pad.reshape(n, cout, ho, w_img)[:, :, :, :wo]
